# Initial kernel scaffold; baseline (speedup 1.0000x reference)
import jax
import jax.numpy as jnp
from jax.experimental import pallas as pl


def kernel(x, W, gamma, beta):
    raise NotImplementedError("write your pallas kernel here")


# speedup vs baseline: 13.1376x; 13.1376x over previous
import functools

import jax
import jax.numpy as jnp
from jax import lax
from jax.experimental import pallas as pl
from jax.experimental.pallas import tpu as pltpu
from jax.experimental.pallas import tpu_sc as plsc

_B, _C, _N, _K, _O = 8, 64, 2048, 16, 64
_BN = _B * _N
_RB = 256
_NEG = float("-inf")

_NC, _NS = 2, 16
_NW = _NC * _NS
_PPW = _BN // _NW
_G = 32
_NGATHER = _G * _K // 128
_NCH = _PPW // _G


def _knn_body(xfull_ref, xrows_ref, w_ref, idx_ref, z12_ref):
    b = pl.program_id(0)
    rb = pl.program_id(1)
    xfull = xfull_ref[0]
    xrows = xrows_ref[0]
    w = w_ref[...]
    w1 = w[:, :_C]
    w2 = w[:, _C:]

    z1 = lax.dot_general(xrows, w1, (((0,), (1,)), ((), ())),
                         preferred_element_type=jnp.float32)
    z2 = lax.dot_general(xrows, w2 - w1, (((0,), (1,)), ((), ())),
                         preferred_element_type=jnp.float32)
    z12_ref[0] = jnp.concatenate([z1, z2], axis=1)

    inner = lax.dot_general(xrows, xfull, (((0,), (0,)), ((), ())),
                            preferred_element_type=jnp.float32)
    sq_full = jnp.sum(xfull * xfull, axis=0, keepdims=True)
    sq_rows = jnp.sum(xrows * xrows, axis=0, keepdims=True)
    dist = (-sq_rows.T - sq_full) + 2.0 * inner

    col = lax.broadcasted_iota(jnp.int32, (_RB, _N), 1)
    row_g = rb * _RB + lax.broadcasted_iota(jnp.int32, (_RB, _N), 0)
    vals = jnp.where(col == row_g, _NEG, dist)

    lane = lax.broadcasted_iota(jnp.int32, (_RB, _K), 1)
    ids = jnp.zeros((_RB, _K), jnp.int32)
    for t in range(_K):
        m = jnp.max(vals, axis=1, keepdims=True)
        cand = jnp.where(vals == m, col, _N)
        a = jnp.min(cand, axis=1, keepdims=True)
        ids = jnp.where(lane == t, a, ids)
        vals = jnp.where(col == a, _NEG, vals)
    idx_ref[0] = ids + b * _N


def _knn_call(x, w):
    return pl.pallas_call(
        _knn_body,
        grid=(_B, _N // _RB),
        in_specs=[
            pl.BlockSpec((1, _C, _N), lambda b, r: (b, 0, 0)),
            pl.BlockSpec((1, _C, _RB), lambda b, r: (b, 0, r)),
            pl.BlockSpec((_O, 2 * _C), lambda b, r: (0, 0)),
        ],
        out_specs=[
            pl.BlockSpec((1, _RB, _K), lambda b, r: (b, r, 0)),
            pl.BlockSpec((1, _RB, 2 * _O), lambda b, r: (b, r, 0)),
        ],
        out_shape=[
            jax.ShapeDtypeStruct((_B, _N, _K), jnp.int32),
            jax.ShapeDtypeStruct((_B, _N, 2 * _O), jnp.float32),
        ],
    )(x, x, w)


def _sc_body(z12_hbm, idx_hbm, ymax_hbm, ymin_hbm, sums_hbm,
             idx_v, rows_v, z12_v, ymax_v, ymin_v, acc_v, sem):
    cid = lax.axis_index("c")
    sid = lax.axis_index("s")
    wid = sid * _NC + cid
    base = wid * _PPW

    zero = jnp.zeros((16,), jnp.float32)
    for g in range(8):
        acc_v[pl.ds(g * 16, 16)] = zero

    @pl.loop(0, _NCH)
    def _chunk(ci):
        p0 = base + ci * _G
        for q in range(_NGATHER):
            pltpu.sync_copy(
                idx_hbm.at[pl.ds(p0 * _K + q * 128, 128)], idx_v.at[q])
        descs = [
            pltpu.async_copy(z12_hbm.at[idx_v.at[q]],
                             rows_v.at[pl.ds(q * 128, 128)], sem)
            for q in range(_NGATHER)
        ]
        pltpu.sync_copy(z12_hbm.at[pl.ds(p0, _G)], z12_v)
        for d in descs:
            d.wait()

        @pl.loop(0, _G)
        def _point(p):
            r0 = p * _K
            for cg in range(_O // 16):
                off = cg * 16
                v = rows_v[r0, pl.ds(off, 16)]
                mx = v
                mn = v
                sm = v
                sq = v * v
                for j in range(1, _K):
                    v = rows_v[r0 + j, pl.ds(off, 16)]
                    mx = jnp.maximum(mx, v)
                    mn = jnp.minimum(mn, v)
                    sm = sm + v
                    sq = sq + v * v
                z2v = z12_v[p, pl.ds(_O + off, 16)]
                ymax_v[p, pl.ds(off, 16)] = mx + z2v
                ymin_v[p, pl.ds(off, 16)] = mn + z2v
                kf = jnp.float32(_K)
                acc_v[pl.ds(off, 16)] += sm + kf * z2v
                acc_v[pl.ds(_O + off, 16)] += (
                    sq + 2.0 * z2v * sm + kf * z2v * z2v)

        pltpu.sync_copy(ymax_v, ymax_hbm.at[pl.ds(p0, _G)])
        pltpu.sync_copy(ymin_v, ymin_hbm.at[pl.ds(p0, _G)])

    pltpu.sync_copy(acc_v, sums_hbm.at[pl.ds(wid * 2 * _O, 2 * _O)])


def _sc_call(z12f, idxf):
    mesh = plsc.VectorSubcoreMesh(
        core_axis_name="c", subcore_axis_name="s",
        num_cores=_NC, num_subcores=_NS)
    f = pl.kernel(
        _sc_body,
        out_type=[
            jax.ShapeDtypeStruct((_BN, _O), jnp.float32),
            jax.ShapeDtypeStruct((_BN, _O), jnp.float32),
            jax.ShapeDtypeStruct((_NW * 2 * _O,), jnp.float32),
        ],
        mesh=mesh,
        scratch_types=[
            pltpu.VMEM((_NGATHER, 128), jnp.int32),
            pltpu.VMEM((_G * _K, 2 * _O), jnp.float32),
            pltpu.VMEM((_G, 2 * _O), jnp.float32),
            pltpu.VMEM((_G, _O), jnp.float32),
            pltpu.VMEM((_G, _O), jnp.float32),
            pltpu.VMEM((2 * _O,), jnp.float32),
            pltpu.SemaphoreType.DMA,
        ],
    )
    return f(z12f, idxf)


def _final_body(ymax_ref, ymin_ref, sums_ref, gamma_ref, beta_ref, out_ref):
    sums = sums_ref[...]
    cnt = jnp.float32(_BN * _K)
    s_a = jnp.sum(sums[:, :_O], axis=0, keepdims=True)
    s_b = jnp.sum(sums[:, _O:], axis=0, keepdims=True)
    mean = s_a / cnt
    var = s_b / cnt - mean * mean
    inv = 1.0 / jnp.sqrt(var + 1e-5)
    slope = gamma_ref[...] * inv
    intercept = beta_ref[...] - mean * slope

    ymax = ymax_ref[0]
    ymin = ymin_ref[0]
    ext = jnp.where(slope >= 0.0, ymax, ymin)
    yn = ext * slope + intercept
    act = jnp.where(yn >= 0.0, yn, 0.2 * yn)

    eye = (lax.broadcasted_iota(jnp.int32, (_O, _O), 0)
           == lax.broadcasted_iota(jnp.int32, (_O, _O), 1)
           ).astype(jnp.float32)
    out_ref[0] = lax.dot_general(eye, act, (((1,), (1,)), ((), ())),
                                 preferred_element_type=jnp.float32)


def _final_call(ymax, ymin, sums, gamma, beta):
    return pl.pallas_call(
        _final_body,
        grid=(_B,),
        in_specs=[
            pl.BlockSpec((1, _N, _O), lambda b: (b, 0, 0)),
            pl.BlockSpec((1, _N, _O), lambda b: (b, 0, 0)),
            pl.BlockSpec((_NW, 2 * _O), lambda b: (0, 0)),
            pl.BlockSpec((1, _O), lambda b: (0, 0)),
            pl.BlockSpec((1, _O), lambda b: (0, 0)),
        ],
        out_specs=pl.BlockSpec((1, _O, _N), lambda b: (b, 0, 0)),
        out_shape=jax.ShapeDtypeStruct((_B, _O, _N), jnp.float32),
    )(ymax, ymin, sums, gamma, beta)


def kernel(x, W, gamma, beta):
    idx, z12 = _knn_call(x, W)
    ymax, ymin, sums = _sc_call(
        z12.reshape(_BN, 2 * _O), idx.reshape(_BN * _K))
    return _final_call(
        ymax.reshape(_B, _N, _O), ymin.reshape(_B, _N, _O),
        sums.reshape(_NW, 2 * _O), gamma.reshape(1, _O), beta.reshape(1, _O))


# speedup vs baseline: 16.3903x; 1.2476x over previous
import functools

import jax
import jax.numpy as jnp
from jax import lax
from jax.experimental import pallas as pl
from jax.experimental.pallas import tpu as pltpu
from jax.experimental.pallas import tpu_sc as plsc

_B, _C, _N, _K, _O = 8, 64, 2048, 16, 64
_BN = _B * _N
_RB = 256
_NEG = float("-inf")

_NC, _NS = 2, 16
_NW = _NC * _NS
_PPW = _BN // _NW
_G = 32
_NGATHER = _G * _K // 128
_NCH = _PPW // _G


def _knn_body(xfull_ref, xrows_ref, w_ref, idx_ref, z12_ref):
    b = pl.program_id(0)
    rb = pl.program_id(1)
    xfull = xfull_ref[0]
    xrows = xrows_ref[0]
    w = w_ref[...]
    w1 = w[:, :_C]
    w2 = w[:, _C:]

    z1 = lax.dot_general(xrows, w1, (((0,), (1,)), ((), ())),
                         preferred_element_type=jnp.float32)
    z2 = lax.dot_general(xrows, w2 - w1, (((0,), (1,)), ((), ())),
                         preferred_element_type=jnp.float32)
    z12_ref[0] = jnp.concatenate([z1, z2], axis=1)

    inner = lax.dot_general(xrows, xfull, (((0,), (0,)), ((), ())),
                            preferred_element_type=jnp.float32)
    sq_full = jnp.sum(xfull * xfull, axis=0, keepdims=True)
    sq_rows = jnp.sum(xrows * xrows, axis=0, keepdims=True)
    dist = (-sq_rows.T - sq_full) + 2.0 * inner

    col = lax.broadcasted_iota(jnp.int32, (_RB, _N), 1)
    row_g = rb * _RB + lax.broadcasted_iota(jnp.int32, (_RB, _N), 0)
    vals = jnp.where(col == row_g, _NEG, dist)

    colf = col.astype(jnp.float32)
    lane = lax.broadcasted_iota(jnp.int32, (_RB, _K), 1)
    idsf = jnp.zeros((_RB, _K), jnp.float32)
    for t in range(_K):
        m = jnp.max(vals, axis=1, keepdims=True)
        cand = jnp.where(vals == m, colf, jnp.float32(_N))
        a = jnp.min(cand, axis=1, keepdims=True)
        idsf = jnp.where(lane == t, a, idsf)
        vals = jnp.where(colf == a, _NEG, vals)
    idx_ref[0] = idsf.astype(jnp.int32) + b * _N


def _knn_call(x, w):
    return pl.pallas_call(
        _knn_body,
        grid=(_B, _N // _RB),
        in_specs=[
            pl.BlockSpec((1, _C, _N), lambda b, r: (b, 0, 0)),
            pl.BlockSpec((1, _C, _RB), lambda b, r: (b, 0, r)),
            pl.BlockSpec((_O, 2 * _C), lambda b, r: (0, 0)),
        ],
        out_specs=[
            pl.BlockSpec((1, _RB, _K), lambda b, r: (b, r, 0)),
            pl.BlockSpec((1, _RB, 2 * _O), lambda b, r: (b, r, 0)),
        ],
        out_shape=[
            jax.ShapeDtypeStruct((_B, _N, _K), jnp.int32),
            jax.ShapeDtypeStruct((_B, _N, 2 * _O), jnp.float32),
        ],
    )(x, x, w)


def _sc_body(z12_hbm, idx_hbm, ymax_hbm, ymin_hbm, sums_hbm,
             idx_v, rows_v, z12_v, ymax_v, ymin_v, acc_v, sem):
    cid = lax.axis_index("c")
    sid = lax.axis_index("s")
    wid = sid * _NC + cid
    base = wid * _PPW

    zero = jnp.zeros((16,), jnp.float32)
    for g in range(8):
        acc_v[pl.ds(g * 16, 16)] = zero

    @pl.loop(0, _NCH)
    def _chunk(ci):
        p0 = base + ci * _G
        for q in range(_NGATHER):
            pltpu.sync_copy(
                idx_hbm.at[pl.ds(p0 * _K + q * 128, 128)], idx_v.at[q])
        descs = [
            pltpu.async_copy(z12_hbm.at[idx_v.at[q]],
                             rows_v.at[pl.ds(q * 128, 128)], sem)
            for q in range(_NGATHER)
        ]
        pltpu.sync_copy(z12_hbm.at[pl.ds(p0, _G)], z12_v)
        for d in descs:
            d.wait()

        @pl.loop(0, _G)
        def _point(p):
            r0 = p * _K
            for cg in range(_O // 16):
                off = cg * 16
                v = rows_v[r0, pl.ds(off, 16)]
                mx = v
                mn = v
                sm = v
                sq = v * v
                for j in range(1, _K):
                    v = rows_v[r0 + j, pl.ds(off, 16)]
                    mx = jnp.maximum(mx, v)
                    mn = jnp.minimum(mn, v)
                    sm = sm + v
                    sq = sq + v * v
                z2v = z12_v[p, pl.ds(_O + off, 16)]
                ymax_v[p, pl.ds(off, 16)] = mx + z2v
                ymin_v[p, pl.ds(off, 16)] = mn + z2v
                kf = jnp.float32(_K)
                acc_v[pl.ds(off, 16)] += sm + kf * z2v
                acc_v[pl.ds(_O + off, 16)] += (
                    sq + 2.0 * z2v * sm + kf * z2v * z2v)

        pltpu.sync_copy(ymax_v, ymax_hbm.at[pl.ds(p0, _G)])
        pltpu.sync_copy(ymin_v, ymin_hbm.at[pl.ds(p0, _G)])

    pltpu.sync_copy(acc_v, sums_hbm.at[pl.ds(wid * 2 * _O, 2 * _O)])


def _sc_call(z12f, idxf):
    mesh = plsc.VectorSubcoreMesh(
        core_axis_name="c", subcore_axis_name="s",
        num_cores=_NC, num_subcores=_NS)
    f = pl.kernel(
        _sc_body,
        out_type=[
            jax.ShapeDtypeStruct((_BN, _O), jnp.float32),
            jax.ShapeDtypeStruct((_BN, _O), jnp.float32),
            jax.ShapeDtypeStruct((_NW * 2 * _O,), jnp.float32),
        ],
        mesh=mesh,
        scratch_types=[
            pltpu.VMEM((_NGATHER, 128), jnp.int32),
            pltpu.VMEM((_G * _K, 2 * _O), jnp.float32),
            pltpu.VMEM((_G, 2 * _O), jnp.float32),
            pltpu.VMEM((_G, _O), jnp.float32),
            pltpu.VMEM((_G, _O), jnp.float32),
            pltpu.VMEM((2 * _O,), jnp.float32),
            pltpu.SemaphoreType.DMA,
        ],
    )
    return f(z12f, idxf)


def _final_body(ymax_ref, ymin_ref, sums_ref, gamma_ref, beta_ref, out_ref):
    sums = sums_ref[...]
    cnt = jnp.float32(_BN * _K)
    s_a = jnp.sum(sums[:, :_O], axis=0, keepdims=True)
    s_b = jnp.sum(sums[:, _O:], axis=0, keepdims=True)
    mean = s_a / cnt
    var = s_b / cnt - mean * mean
    inv = 1.0 / jnp.sqrt(var + 1e-5)
    slope = gamma_ref[...] * inv
    intercept = beta_ref[...] - mean * slope

    ymax = ymax_ref[0]
    ymin = ymin_ref[0]
    ext = jnp.where(slope >= 0.0, ymax, ymin)
    yn = ext * slope + intercept
    act = jnp.where(yn >= 0.0, yn, 0.2 * yn)

    eye = (lax.broadcasted_iota(jnp.int32, (_O, _O), 0)
           == lax.broadcasted_iota(jnp.int32, (_O, _O), 1)
           ).astype(jnp.float32)
    out_ref[0] = lax.dot_general(eye, act, (((1,), (1,)), ((), ())),
                                 preferred_element_type=jnp.float32)


def _final_call(ymax, ymin, sums, gamma, beta):
    return pl.pallas_call(
        _final_body,
        grid=(_B,),
        in_specs=[
            pl.BlockSpec((1, _N, _O), lambda b: (b, 0, 0)),
            pl.BlockSpec((1, _N, _O), lambda b: (b, 0, 0)),
            pl.BlockSpec((_NW, 2 * _O), lambda b: (0, 0)),
            pl.BlockSpec((1, _O), lambda b: (0, 0)),
            pl.BlockSpec((1, _O), lambda b: (0, 0)),
        ],
        out_specs=pl.BlockSpec((1, _O, _N), lambda b: (b, 0, 0)),
        out_shape=jax.ShapeDtypeStruct((_B, _O, _N), jnp.float32),
    )(ymax, ymin, sums, gamma, beta)


def kernel(x, W, gamma, beta):
    idx, z12 = _knn_call(x, W)
    ymax, ymin, sums = _sc_call(
        z12.reshape(_BN, 2 * _O), idx.reshape(_BN * _K))
    return _final_call(
        ymax.reshape(_B, _N, _O), ymin.reshape(_B, _N, _O),
        sums.reshape(_NW, 2 * _O), gamma.reshape(1, _O), beta.reshape(1, _O))
